# baseline (speedup 1.0000x reference)
import functools

import jax
import jax.numpy as jnp
from jax import lax
from jax.experimental import pallas as pl
from jax.experimental.pallas import tpu as pltpu
from jax.experimental.pallas import tpu_sc as plsc

N_COLS = 22
VOCAB = 100000
EMB = 32
BATCH = 16384
TOTAL = BATCH * N_COLS
PACKED_ROWS = N_COLS * VOCAB // 4
NUM_WORKERS = 32
PER_WORKER = TOTAL // NUM_WORKERS
CHUNK = 256
NUM_CHUNKS = PER_WORKER // CHUNK
GROUPS = CHUNK // 16

assert TOTAL % NUM_WORKERS == 0
assert PER_WORKER % CHUNK == 0

_mesh = plsc.VectorSubcoreMesh(core_axis_name="c", subcore_axis_name="s")


@functools.partial(
    pl.kernel,
    mesh=_mesh,
    out_type=jax.ShapeDtypeStruct((TOTAL // 2, 128), jnp.float32),
    scratch_types=[
        pltpu.VMEM((CHUNK,), jnp.int32),
        pltpu.VMEM((CHUNK,), jnp.int32),
        pltpu.VMEM((CHUNK, 128), jnp.float32),
        pltpu.VMEM((CHUNK, 128), jnp.float32),
        pltpu.VMEM((CHUNK // 2, 128), jnp.float32),
        pltpu.SemaphoreType.DMA,
        pltpu.SemaphoreType.DMA,
    ],
)
def _gather_kernel(idx_hbm, mean_hbm, std_hbm, out_hbm,
                   idx_v, idx128_v, mean_v, std_v, out_v, sem_m, sem_s):
    wid = lax.axis_index("s") * 2 + lax.axis_index("c")
    base = wid * PER_WORKER

    def chunk_body(i, carry):
        off = base + i * CHUNK
        pltpu.sync_copy(idx_hbm.at[pl.ds(off, CHUNK)], idx_v)

        def packed_body(g, c):
            fvec = idx_v[pl.ds(g * 16, 16)]
            idx128_v[pl.ds(g * 16, 16)] = lax.shift_right_logical(fvec, 2)
            return c

        lax.fori_loop(0, GROUPS, packed_body, 0)

        cm = pltpu.async_copy(mean_hbm.at[idx128_v], mean_v, sem_m)
        cs = pltpu.async_copy(std_hbm.at[idx128_v], std_v, sem_s)
        cm.wait()
        cs.wait()

        def extract_body(g, c):
            i0 = g * 16
            csvec = (idx_v[pl.ds(i0, 16)] & 3) * 32
            for l in range(16):
                t = i0 + l
                srcc = csvec[l]
                dr = lax.shift_right_logical(t, 1)
                dc = (l & 1) * 64
                out_v[dr, pl.ds(dc, 16)] = mean_v[t, pl.ds(srcc, 16)]
                out_v[dr, pl.ds(dc + 16, 16)] = mean_v[t, pl.ds(srcc + 16, 16)]
                out_v[dr, pl.ds(dc + 32, 16)] = std_v[t, pl.ds(srcc, 16)]
                out_v[dr, pl.ds(dc + 48, 16)] = std_v[t, pl.ds(srcc + 16, 16)]
            return c

        lax.fori_loop(0, GROUPS, extract_body, 0)
        out_off = pl.multiple_of(off // 2, 8)
        pltpu.sync_copy(out_v, out_hbm.at[pl.ds(out_off, CHUNK // 2)])
        return carry

    lax.fori_loop(0, NUM_CHUNKS, chunk_body, 0)


def kernel(features, emb_mean, emb_std):
    flat_idx = (features.astype(jnp.int32)
                + (jnp.arange(N_COLS, dtype=jnp.int32) * VOCAB)[None, :])
    flat_idx = flat_idx.reshape(TOTAL)
    mean2d = emb_mean.reshape(PACKED_ROWS, 128)
    std2d = emb_std.reshape(PACKED_ROWS, 128)
    out = _gather_kernel(flat_idx, mean2d, std2d)
    return out.reshape(BATCH, N_COLS, 2 * EMB)


# speedup vs baseline: 1.1411x; 1.1411x over previous
import functools

import jax
import jax.numpy as jnp
from jax import lax
from jax.experimental import pallas as pl
from jax.experimental.pallas import tpu as pltpu
from jax.experimental.pallas import tpu_sc as plsc

N_COLS = 22
VOCAB = 100000
EMB = 32
BATCH = 16384
TOTAL = BATCH * N_COLS
NUM_WORKERS = 32
PER_WORKER = TOTAL // NUM_WORKERS
CHUNK = 512
NUM_CHUNKS = PER_WORKER // CHUNK

assert TOTAL % NUM_WORKERS == 0
assert PER_WORKER % CHUNK == 0

_mesh = plsc.VectorSubcoreMesh(core_axis_name="c", subcore_axis_name="s")


@functools.partial(
    pl.kernel,
    mesh=_mesh,
    compiler_params=pltpu.CompilerParams(use_tc_tiling_on_sc=False),
    out_type=jax.ShapeDtypeStruct((TOTAL, 2, EMB), jnp.float32),
    scratch_types=[
        pltpu.VMEM((2, CHUNK), jnp.int32),
        pltpu.VMEM((2, CHUNK, EMB), jnp.float32),
        pltpu.VMEM((2, CHUNK, EMB), jnp.float32),
        pltpu.SemaphoreType.DMA,
        pltpu.SemaphoreType.DMA,
        pltpu.SemaphoreType.DMA,
        pltpu.SemaphoreType.DMA,
        pltpu.SemaphoreType.DMA,
        pltpu.SemaphoreType.DMA,
    ],
)
def _gather_kernel(idx_hbm, mean_hbm, std_hbm, out_hbm,
                   idx_v, mean_v, std_v, sm0, sm1, ss0, ss1, so0, so1):
    wid = lax.axis_index("s") * 2 + lax.axis_index("c")
    base = wid * PER_WORKER
    mean2d = mean_hbm
    std2d = std_hbm
    sem_m = (sm0, sm1)
    sem_s = (ss0, ss1)
    sem_o = (so0, so1)

    def start_chunk(i, b):
        off = base + i * CHUNK
        pltpu.sync_copy(idx_hbm.at[pl.ds(off, CHUNK)], idx_v.at[b])
        cm = pltpu.async_copy(mean2d.at[idx_v.at[b]], mean_v.at[b], sem_m[b])
        cs = pltpu.async_copy(std2d.at[idx_v.at[b]], std_v.at[b], sem_s[b])
        return cm, cs

    gathers = {0: start_chunk(0, 0)}
    outs = {}
    for i in range(NUM_CHUNKS):
        b = i & 1
        nb = 1 - b
        if i + 1 < NUM_CHUNKS:
            if i - 1 >= 0:
                for c in outs.pop(i - 1):
                    c.wait()
            gathers[i + 1] = start_chunk(i + 1, nb)
        cm, cs = gathers.pop(i)
        cm.wait()
        cs.wait()
        off = base + i * CHUNK
        om = pltpu.async_copy(mean_v.at[b], out_hbm.at[pl.ds(off, CHUNK), 0],
                              sem_o[b])
        os_ = pltpu.async_copy(std_v.at[b], out_hbm.at[pl.ds(off, CHUNK), 1],
                               sem_o[b])
        outs[i] = (om, os_)
    for i in sorted(outs):
        for c in outs[i]:
            c.wait()


def kernel(features, emb_mean, emb_std):
    flat_idx = (features.astype(jnp.int32)
                + (jnp.arange(N_COLS, dtype=jnp.int32) * VOCAB)[None, :])
    flat_idx = flat_idx.reshape(TOTAL)
    mean2d = emb_mean.reshape(N_COLS * VOCAB, EMB)
    std2d = emb_std.reshape(N_COLS * VOCAB, EMB)
    out = _gather_kernel(flat_idx, mean2d, std2d)
    return out.reshape(BATCH, N_COLS, 2 * EMB)


# speedup vs baseline: 1.1420x; 1.0008x over previous
import functools

import jax
import jax.numpy as jnp
from jax import lax
from jax.experimental import pallas as pl
from jax.experimental.pallas import tpu as pltpu
from jax.experimental.pallas import tpu_sc as plsc

N_COLS = 22
VOCAB = 100000
EMB = 32
BATCH = 16384
TOTAL = BATCH * N_COLS
NUM_WORKERS = 32
PER_WORKER = TOTAL // NUM_WORKERS
CHUNK = 512
NUM_CHUNKS = PER_WORKER // CHUNK

assert TOTAL % NUM_WORKERS == 0
assert PER_WORKER % CHUNK == 0

_mesh = plsc.VectorSubcoreMesh(core_axis_name="c", subcore_axis_name="s")


@functools.partial(
    pl.kernel,
    mesh=_mesh,
    compiler_params=pltpu.CompilerParams(use_tc_tiling_on_sc=False),
    out_type=jax.ShapeDtypeStruct((TOTAL, 2, EMB), jnp.float32),
    scratch_types=[
        pltpu.VMEM((2, CHUNK), jnp.int32),
        pltpu.VMEM((2, CHUNK, EMB), jnp.float32),
        pltpu.VMEM((2, CHUNK, EMB), jnp.float32),
        pltpu.SemaphoreType.DMA,
        pltpu.SemaphoreType.DMA,
        pltpu.SemaphoreType.DMA,
        pltpu.SemaphoreType.DMA,
        pltpu.SemaphoreType.DMA,
        pltpu.SemaphoreType.DMA,
    ],
)
def _gather_kernel(idx_hbm, mean_hbm, std_hbm, out_hbm,
                   idx_v, mean_v, std_v, sm0, sm1, ss0, ss1, so0, so1):
    wid = lax.axis_index("s") * 2 + lax.axis_index("c")
    base = wid * PER_WORKER
    sem_m = (sm0, sm1)
    sem_s = (ss0, ss1)
    sem_o = (so0, so1)

    def start_chunk(i, b):
        off = base + i * CHUNK
        pltpu.sync_copy(idx_hbm.at[pl.ds(off, CHUNK)], idx_v.at[b])
        cm = pltpu.async_copy(mean_hbm.at[idx_v.at[b]], mean_v.at[b], sem_m[b])
        cs = pltpu.async_copy(std_hbm.at[idx_v.at[b]], std_v.at[b], sem_s[b])
        return cm, cs

    gathers = {0: start_chunk(0, 0)}
    outs = {}
    for i in range(NUM_CHUNKS):
        b = i & 1
        nb = 1 - b
        if i + 1 < NUM_CHUNKS:
            if i - 1 >= 0:
                for c in outs.pop(i - 1):
                    c.wait()
            gathers[i + 1] = start_chunk(i + 1, nb)
        cm, cs = gathers.pop(i)
        cm.wait()
        cs.wait()
        off = base + i * CHUNK
        om = pltpu.async_copy(mean_v.at[b], out_hbm.at[pl.ds(off, CHUNK), 0],
                              sem_o[b])
        os_ = pltpu.async_copy(std_v.at[b], out_hbm.at[pl.ds(off, CHUNK), 1],
                               sem_o[b])
        outs[i] = (om, os_)
    for i in sorted(outs):
        for c in outs[i]:
            c.wait()


def kernel(features, emb_mean, emb_std):
    flat_idx = (features.astype(jnp.int32)
                + (jnp.arange(N_COLS, dtype=jnp.int32) * VOCAB)[None, :])
    flat_idx = flat_idx.reshape(TOTAL)
    mean2d = emb_mean.reshape(N_COLS * VOCAB, EMB)
    std2d = emb_std.reshape(N_COLS * VOCAB, EMB)
    out = _gather_kernel(flat_idx, mean2d, std2d)
    return out.reshape(BATCH, N_COLS, 2 * EMB)
